# Initial kernel scaffold; baseline (speedup 1.0000x reference)
import jax
import jax.numpy as jnp
from jax.experimental import pallas as pl


def kernel(center_words, target_words, neg_words, V_w, U_w):
    raise NotImplementedError("write your pallas kernel here")


# speedup vs baseline: 1.0672x; 1.0672x over previous
import functools

import jax
import jax.numpy as jnp
from jax import lax
from jax.experimental import pallas as pl
from jax.experimental.pallas import tpu as pltpu
from jax.experimental.pallas import tpu_sc as plsc

_B = 4096
_D = 64
_L = 16
_NC, _NS = 2, 16
_NW = _NC * _NS
_BPW = _B // _NW
_C = 32
_NCHUNK = _BPW // _C
_NSCORE = 21
_UROWS = _NSCORE * _C
_UIW = 84
_UIR = _UROWS // _UIW


def _sc_body(cidx_hbm, uidx_hbm, v_hbm, u_hbm, out_hbm,
             cidx_v, uidx_v, vrows, urows, parts, sem):
    wid = lax.axis_index("s") * _NC + lax.axis_index("c")
    for g in range(_NCHUNK):
        base = wid * _BPW + g * _C
        irow = (wid * _NCHUNK + g) * _UIR
        pltpu.sync_copy(cidx_hbm.at[pl.ds(base * 4, _C * 4)], cidx_v)
        pltpu.sync_copy(uidx_hbm.at[pl.ds(irow, _UIR)], uidx_v)
        cps = [pltpu.async_copy(v_hbm.at[cidx_v], vrows, sem)]
        for i in range(_UIR):
            cps.append(pltpu.async_copy(
                u_hbm.at[uidx_v.at[i]],
                urows.at[pl.ds(i * _UIW, _UIW)], sem))
        for cp in cps:
            cp.wait()

        def elem(c, carry):
            v = []
            for k in range(4):
                sl = pl.ds(16 * k, 16)
                v.append((vrows[4 * c, sl] + vrows[4 * c + 1, sl]
                          + vrows[4 * c + 2, sl] + vrows[4 * c + 3, sl])
                         * 0.25)
            row = _NSCORE * c
            for j in range(_NSCORE):
                p = urows[row + j, pl.ds(0, 16)] * v[0]
                for k in range(1, 4):
                    p = p + urows[row + j, pl.ds(16 * k, 16)] * v[k]
                parts[c, pl.ds(16 * j, 16)] = p if j == 0 else -p
            return carry

        lax.fori_loop(0, _C, elem, 0)
        pltpu.sync_copy(parts, out_hbm.at[pl.ds(base, _C)])


_sc_call = functools.partial(
    pl.kernel,
    out_type=jax.ShapeDtypeStruct((_B, _NSCORE * _L), jnp.float32),
    mesh=plsc.VectorSubcoreMesh(core_axis_name="c", subcore_axis_name="s"),
    scratch_types=[
        pltpu.VMEM((_C * 4,), jnp.int32),
        pltpu.VMEM((_UIR, _UIW), jnp.int32),
        pltpu.VMEM((_C * 4, _D), jnp.float32),
        pltpu.VMEM((_UROWS, _D), jnp.float32),
        pltpu.VMEM((_C, _NSCORE * _L), jnp.float32),
        pltpu.SemaphoreType.DMA,
    ],
    compiler_params=pltpu.CompilerParams(use_tc_tiling_on_sc=False),
)(_sc_body)


def _tc_body(x_ref, o_ref):
    x = x_ref[...]
    ii = lax.broadcasted_iota(jnp.int32, (_NSCORE * _L, _NSCORE), 0)
    jj = lax.broadcasted_iota(jnp.int32, (_NSCORE * _L, _NSCORE), 1)
    m = jnp.where(ii // _L == jj, 1.0, 0.0)
    s = jnp.dot(x, m, preferred_element_type=jnp.float32)
    ls = jnp.minimum(s, 0.0) - jnp.log1p(jnp.exp(-jnp.abs(s)))
    o_ref[...] = jnp.full((1, 1), -jnp.sum(ls) / _B, jnp.float32)


_tc_call = pl.pallas_call(
    _tc_body,
    out_shape=jax.ShapeDtypeStruct((1, 1), jnp.float32),
)


def kernel(center_words, target_words, neg_words, V_w, U_w):
    cidx = center_words.astype(jnp.int32).reshape(-1)
    uidx = jnp.concatenate(
        [target_words.astype(jnp.int32), neg_words.astype(jnp.int32)],
        axis=1).reshape(_NW * _NCHUNK * _UIR, _UIW)
    parts = _sc_call(cidx, uidx, V_w, U_w)
    loss = _tc_call(parts)
    return loss[0, 0]


# speedup vs baseline: 1.3041x; 1.2220x over previous
import functools

import jax
import jax.numpy as jnp
from jax import lax
from jax.experimental import pallas as pl
from jax.experimental.pallas import tpu as pltpu
from jax.experimental.pallas import tpu_sc as plsc

_B = 4096
_V = 100000
_D = 64
_L = 16
_NC, _NS = 2, 16
_NW = _NC * _NS
_BPW = _B // _NW
_C = 32
_NCHUNK = _BPW // _C
_NSCORE = 21
_UROWS = _NSCORE * _C
_UIW = 96
_UIR = _UROWS // _UIW


def _sc_body(cidx_hbm, uidx_hbm, v_hbm, u_hbm, out_hbm,
             cidx_v, uidx_v, vrows, urows, out_v, sem0, sem1):
    sems = (sem0, sem1)
    wid = lax.axis_index("s") * _NC + lax.axis_index("c")

    def issue(g, b):
        base = wid * _BPW + g * _C
        pltpu.sync_copy(cidx_hbm.at[pl.ds(base * 4, _C * 4)], cidx_v.at[b])
        pltpu.sync_copy(uidx_hbm.at[pl.ds(base * _NSCORE, _UROWS)],
                        uidx_v.at[b])
        cps = [pltpu.async_copy(v_hbm.at[cidx_v.at[b]], vrows.at[b],
                                sems[b])]
        for i in range(_UIR):
            sl = pl.ds(i * _UIW, _UIW)
            cps.append(pltpu.async_copy(u_hbm.at[uidx_v.at[b, sl]],
                                        urows.at[b, sl], sems[b]))
        return cps

    lanes = lax.iota(jnp.int32, _L)
    cps = issue(0, 0)
    for g in range(_NCHUNK):
        b = g % 2
        nxt = issue(g + 1, 1 - b) if g + 1 < _NCHUNK else []
        for cp in cps:
            cp.wait()
        cps = nxt

        def elem(c, carry, b=b):
            sl = [pl.ds(16 * k, 16) for k in range(4)]
            v = [(vrows[b, 4 * c, s] + vrows[b, 4 * c + 1, s]
                  + vrows[b, 4 * c + 2, s] + vrows[b, 4 * c + 3, s]) * 0.25
                 for s in sl]
            acc0 = jnp.zeros((_L,), jnp.float32)
            acc1 = jnp.zeros((_L,), jnp.float32)
            row = _NSCORE * c
            for j in range(_NSCORE):
                p = urows[b, row + j, sl[0]] * v[0]
                for k in range(1, 4):
                    p = p + urows[b, row + j, sl[k]] * v[k]
                t = jnp.sum(p)
                t = t if j == 0 else -t
                if j < _L:
                    acc0 = jnp.where(lanes == j, t, acc0)
                else:
                    acc1 = jnp.where(lanes == (j - _L), t, acc1)
            out_v[c, pl.ds(0, _L)] = acc0
            out_v[c, pl.ds(_L, _L)] = acc1
            return carry

        lax.fori_loop(0, _C, elem, 0)
        base = wid * _BPW + g * _C
        pltpu.sync_copy(out_v, out_hbm.at[pl.ds(base, _C)])


_sc_call = functools.partial(
    pl.kernel,
    out_type=jax.ShapeDtypeStruct((_B, 2 * _L), jnp.float32),
    mesh=plsc.VectorSubcoreMesh(core_axis_name="c", subcore_axis_name="s"),
    scratch_types=[
        pltpu.VMEM((2, _C * 4), jnp.int32),
        pltpu.VMEM((2, _UROWS), jnp.int32),
        pltpu.VMEM((2, _C * 4, _D), jnp.float32),
        pltpu.VMEM((2, _UROWS, _D), jnp.float32),
        pltpu.VMEM((_C, 2 * _L), jnp.float32),
        pltpu.SemaphoreType.DMA,
        pltpu.SemaphoreType.DMA,
    ],
    compiler_params=pltpu.CompilerParams(use_tc_tiling_on_sc=False,
                                         needs_layout_passes=False),
)(_sc_body)


def _tc_body(x_ref, o_ref):
    x = x_ref[...]
    col = lax.broadcasted_iota(jnp.int32, (_B, 2 * _L), 1)
    ls = jnp.minimum(x, 0.0) - jnp.log1p(jnp.exp(-jnp.abs(x)))
    ls = jnp.where(col < _NSCORE, ls, 0.0)
    o_ref[...] = jnp.full((1, 1), -jnp.sum(ls) / _B, jnp.float32)


_tc_call = pl.pallas_call(
    _tc_body,
    out_shape=jax.ShapeDtypeStruct((1, 1), jnp.float32),
)


def kernel(center_words, target_words, neg_words, V_w, U_w):
    cidx = center_words.astype(jnp.int32).reshape(-1)
    uidx = jnp.concatenate(
        [target_words.astype(jnp.int32), neg_words.astype(jnp.int32)],
        axis=1).reshape(-1)
    scores = _sc_call(cidx, uidx, V_w, U_w)
    loss = _tc_call(scores)
    return loss[0, 0]
